# baseline (speedup 1.0000x reference)
import functools

import jax
import jax.numpy as jnp
from jax import lax
from jax.experimental import pallas as pl
from jax.experimental.pallas import tpu as pltpu
from jax.experimental.pallas import tpu_sc as plsc

_LANES = 16


def _make_sc_kernel(B, SEQ):
    info = plsc.get_sparse_core_info()
    NC, NS = info.num_cores, info.num_subcores
    NW = NC * NS
    R = B // NW
    N = R * SEQ
    G = R // _LANES
    assert R % _LANES == 0 and B % NW == 0
    mesh = plsc.VectorSubcoreMesh(core_axis_name="c", subcore_axis_name="s")

    def _build_tidx(idx_v, tidx_v):
        iota = lax.iota(jnp.int32, _LANES)

        def body(j, _):
            cols = jnp.full((_LANES,), 0, jnp.int32) + j
            for g in range(G):
                rows = g * _LANES + iota
                tidx_v[pl.ds(j * R + g * _LANES, _LANES)] = plsc.load_gather(
                    idx_v, [rows, cols]
                )
            return 0

        lax.fori_loop(0, SEQ, body, 0, unroll=False)

    def _gather(w_hbm, tidx_v, vals_v, sem):
        def issue(j, _):
            pltpu.async_copy(
                w_hbm.at[tidx_v.at[pl.ds(j * R, R)]],
                vals_v.at[pl.ds(j * R, R)],
                sem,
            )
            return 0

        lax.fori_loop(0, SEQ, issue, 0, unroll=False)

    def _drain(w_hbm, tidx_v, vals_v, sem):
        def body(j, _):
            pltpu.make_async_copy(
                w_hbm.at[tidx_v.at[pl.ds(j * R, R)]],
                vals_v.at[pl.ds(j * R, R)],
                sem,
            ).wait()
            return 0

        lax.fori_loop(0, SEQ, body, 0, unroll=False)

    def _reduce(vals_v, out_v):
        def body(j, accs):
            return tuple(
                accs[g] + vals_v[pl.ds(j * R + g * _LANES, _LANES)]
                for g in range(G)
            )

        zeros = jnp.zeros((_LANES,), jnp.float32)
        accs = lax.fori_loop(0, SEQ, body, (zeros,) * G, unroll=False)
        for g in range(G):
            out_v[pl.ds(g * _LANES, _LANES)] = accs[g]

    @functools.partial(
        pl.kernel,
        out_type=(
            jax.ShapeDtypeStruct((B,), jnp.float32),
            jax.ShapeDtypeStruct((B,), jnp.float32),
        ),
        mesh=mesh,
        compiler_params=pltpu.CompilerParams(needs_layout_passes=False),
        scratch_types=dict(
            idx_p=pltpu.VMEM((R, SEQ), jnp.int32),
            idx_s=pltpu.VMEM((R, SEQ), jnp.int32),
            tidx_p=pltpu.VMEM((N,), jnp.int32),
            tidx_s=pltpu.VMEM((N,), jnp.int32),
            vals_p=pltpu.VMEM((N,), jnp.float32),
            vals_s=pltpu.VMEM((N,), jnp.float32),
            out_p=pltpu.VMEM((R,), jnp.float32),
            out_s=pltpu.VMEM((R,), jnp.float32),
            sem_p=pltpu.SemaphoreType.DMA,
            sem_s=pltpu.SemaphoreType.DMA,
        ),
    )
    def sc_kernel(
        idxp_hbm,
        idxs_hbm,
        wp_hbm,
        ws_hbm,
        outp_hbm,
        outs_hbm,
        *,
        idx_p,
        idx_s,
        tidx_p,
        tidx_s,
        vals_p,
        vals_s,
        out_p,
        out_s,
        sem_p,
        sem_s,
    ):
        wid = lax.axis_index("s") * NC + lax.axis_index("c")
        rbase = wid * R

        pltpu.sync_copy(idxp_hbm.at[pl.ds(rbase, R), :], idx_p)
        _build_tidx(idx_p, tidx_p)
        _gather(wp_hbm, tidx_p, vals_p, sem_p)

        pltpu.sync_copy(idxs_hbm.at[pl.ds(rbase, R), :], idx_s)
        _build_tidx(idx_s, tidx_s)
        _gather(ws_hbm, tidx_s, vals_s, sem_s)

        _drain(wp_hbm, tidx_p, vals_p, sem_p)
        _reduce(vals_p, out_p)
        pltpu.sync_copy(out_p, outp_hbm.at[pl.ds(rbase, R)])

        _drain(ws_hbm, tidx_s, vals_s, sem_s)
        _reduce(vals_s, out_s)
        pltpu.sync_copy(out_s, outs_hbm.at[pl.ds(rbase, R)])

    return sc_kernel


def kernel(inputs_pri, inputs_sec, W_pri, W_sec):
    B, SEQ = inputs_pri.shape
    sc = _make_sc_kernel(B, SEQ)
    out_pri, out_sec = sc(
        inputs_pri.astype(jnp.int32),
        inputs_sec.astype(jnp.int32),
        W_pri[:, 0],
        W_sec[:, 0],
    )
    return (out_pri, out_sec)


# speedup vs baseline: 1.0513x; 1.0513x over previous
import functools

import jax
import jax.numpy as jnp
from jax import lax
from jax.experimental import pallas as pl
from jax.experimental.pallas import tpu as pltpu
from jax.experimental.pallas import tpu_sc as plsc

_LANES = 16


def _make_sc_kernel(B, SEQ):
    info = plsc.get_sparse_core_info()
    NC, NS = info.num_cores, info.num_subcores
    NW = NC * NS
    R = B // NW
    N = R * SEQ
    G = R // _LANES
    CH = 128
    NDMA = N // CH
    assert R % _LANES == 0 and B % NW == 0 and N % CH == 0
    mesh = plsc.VectorSubcoreMesh(core_axis_name="c", subcore_axis_name="s")

    def _fire(w_hbm, idx_v, vals_v, sem):
        def issue(c, _):
            pltpu.async_copy(
                w_hbm.at[idx_v.at[pl.ds(c * CH, CH)]],
                vals_v.at[pl.ds(c * CH, CH)],
                sem,
            )
            return 0

        lax.fori_loop(0, NDMA, issue, 0, unroll=False)

    def _drain(w_hbm, idx_v, vals_v, sem):
        def body(c, _):
            pltpu.make_async_copy(
                w_hbm.at[idx_v.at[pl.ds(c * CH, CH)]],
                vals_v.at[pl.ds(c * CH, CH)],
                sem,
            ).wait()
            return 0

        lax.fori_loop(0, NDMA, body, 0, unroll=False)

    def _reduce(vals_v, out_v):
        iota_rows = lax.iota(jnp.int32, _LANES) * SEQ

        def body(j, accs):
            return tuple(
                accs[g]
                + plsc.load_gather(vals_v, [iota_rows + (g * _LANES * SEQ + j)])
                for g in range(G)
            )

        zeros = jnp.zeros((_LANES,), jnp.float32)
        accs = lax.fori_loop(0, SEQ, body, (zeros,) * G, unroll=False)
        for g in range(G):
            out_v[pl.ds(g * _LANES, _LANES)] = accs[g]

    @functools.partial(
        pl.kernel,
        out_type=(
            jax.ShapeDtypeStruct((B,), jnp.float32),
            jax.ShapeDtypeStruct((B,), jnp.float32),
        ),
        mesh=mesh,
        compiler_params=pltpu.CompilerParams(needs_layout_passes=False),
        scratch_types=dict(
            idx_p=pltpu.VMEM((N,), jnp.int32),
            idx_s=pltpu.VMEM((N,), jnp.int32),
            vals_p=pltpu.VMEM((N,), jnp.float32),
            vals_s=pltpu.VMEM((N,), jnp.float32),
            out_p=pltpu.VMEM((R,), jnp.float32),
            out_s=pltpu.VMEM((R,), jnp.float32),
            sem_p=pltpu.SemaphoreType.DMA,
            sem_s=pltpu.SemaphoreType.DMA,
        ),
    )
    def sc_kernel(
        idxp_hbm,
        idxs_hbm,
        wp_hbm,
        ws_hbm,
        outp_hbm,
        outs_hbm,
        *,
        idx_p,
        idx_s,
        vals_p,
        vals_s,
        out_p,
        out_s,
        sem_p,
        sem_s,
    ):
        wid = lax.axis_index("s") * NC + lax.axis_index("c")
        base = wid * N
        rbase = wid * R

        pltpu.sync_copy(idxp_hbm.at[pl.ds(base, N)], idx_p)
        _fire(wp_hbm, idx_p, vals_p, sem_p)

        pltpu.sync_copy(idxs_hbm.at[pl.ds(base, N)], idx_s)
        _fire(ws_hbm, idx_s, vals_s, sem_s)

        _drain(wp_hbm, idx_p, vals_p, sem_p)
        _reduce(vals_p, out_p)
        pltpu.sync_copy(out_p, outp_hbm.at[pl.ds(rbase, R)])

        _drain(ws_hbm, idx_s, vals_s, sem_s)
        _reduce(vals_s, out_s)
        pltpu.sync_copy(out_s, outs_hbm.at[pl.ds(rbase, R)])

    return sc_kernel


def kernel(inputs_pri, inputs_sec, W_pri, W_sec):
    B, SEQ = inputs_pri.shape
    sc = _make_sc_kernel(B, SEQ)
    out_pri, out_sec = sc(
        inputs_pri.reshape(-1).astype(jnp.int32),
        inputs_sec.reshape(-1).astype(jnp.int32),
        W_pri[:, 0],
        W_sec[:, 0],
    )
    return (out_pri, out_sec)


# speedup vs baseline: 1.0587x; 1.0070x over previous
import functools

import jax
import jax.numpy as jnp
from jax import lax
from jax.experimental import pallas as pl
from jax.experimental.pallas import tpu as pltpu
from jax.experimental.pallas import tpu_sc as plsc

_LANES = 16


def _make_sc_kernel(B, SEQ):
    info = plsc.get_sparse_core_info()
    NC, NS = info.num_cores, info.num_subcores
    NW = NC * NS
    R = B // NW
    N = R * SEQ
    G = R // _LANES
    CH = 128
    NDMA = N // CH
    assert R % _LANES == 0 and B % NW == 0 and N % CH == 0
    mesh = plsc.VectorSubcoreMesh(core_axis_name="c", subcore_axis_name="s")

    def _fire(w_hbm, idx_v, vals_v, sem):
        pltpu.async_copy(w_hbm.at[idx_v], vals_v, sem)

    def _drain(w_hbm, idx_v, vals_v, sem):
        pltpu.make_async_copy(w_hbm.at[idx_v], vals_v, sem).wait()

    def _reduce(vals_v, out_v):
        iota_rows = lax.iota(jnp.int32, _LANES) * SEQ

        def body(j, accs):
            return tuple(
                accs[g]
                + plsc.load_gather(vals_v, [iota_rows + (g * _LANES * SEQ + j)])
                for g in range(G)
            )

        zeros = jnp.zeros((_LANES,), jnp.float32)
        accs = lax.fori_loop(0, SEQ, body, (zeros,) * G, unroll=2)
        for g in range(G):
            out_v[pl.ds(g * _LANES, _LANES)] = accs[g]

    @functools.partial(
        pl.kernel,
        out_type=(
            jax.ShapeDtypeStruct((B,), jnp.float32),
            jax.ShapeDtypeStruct((B,), jnp.float32),
        ),
        mesh=mesh,
        compiler_params=pltpu.CompilerParams(needs_layout_passes=False),
        scratch_types=dict(
            idx_p=pltpu.VMEM((N,), jnp.int32),
            idx_s=pltpu.VMEM((N,), jnp.int32),
            vals_p=pltpu.VMEM((N,), jnp.float32),
            vals_s=pltpu.VMEM((N,), jnp.float32),
            out_p=pltpu.VMEM((R,), jnp.float32),
            out_s=pltpu.VMEM((R,), jnp.float32),
            sem_p=pltpu.SemaphoreType.DMA,
            sem_s=pltpu.SemaphoreType.DMA,
        ),
    )
    def sc_kernel(
        idxp_hbm,
        idxs_hbm,
        wp_hbm,
        ws_hbm,
        outp_hbm,
        outs_hbm,
        *,
        idx_p,
        idx_s,
        vals_p,
        vals_s,
        out_p,
        out_s,
        sem_p,
        sem_s,
    ):
        wid = lax.axis_index("s") * NC + lax.axis_index("c")
        base = wid * N
        rbase = wid * R

        pltpu.sync_copy(idxp_hbm.at[pl.ds(base, N)], idx_p)
        _fire(wp_hbm, idx_p, vals_p, sem_p)

        pltpu.sync_copy(idxs_hbm.at[pl.ds(base, N)], idx_s)
        _fire(ws_hbm, idx_s, vals_s, sem_s)

        _drain(wp_hbm, idx_p, vals_p, sem_p)
        _reduce(vals_p, out_p)
        pltpu.sync_copy(out_p, outp_hbm.at[pl.ds(rbase, R)])

        _drain(ws_hbm, idx_s, vals_s, sem_s)
        _reduce(vals_s, out_s)
        pltpu.sync_copy(out_s, outs_hbm.at[pl.ds(rbase, R)])

    return sc_kernel


def kernel(inputs_pri, inputs_sec, W_pri, W_sec):
    B, SEQ = inputs_pri.shape
    sc = _make_sc_kernel(B, SEQ)
    out_pri, out_sec = sc(
        inputs_pri.reshape(-1).astype(jnp.int32),
        inputs_sec.reshape(-1).astype(jnp.int32),
        W_pri[:, 0],
        W_sec[:, 0],
    )
    return (out_pri, out_sec)


# speedup vs baseline: 1.0652x; 1.0062x over previous
import functools

import jax
import jax.numpy as jnp
from jax import lax
from jax.experimental import pallas as pl
from jax.experimental.pallas import tpu as pltpu
from jax.experimental.pallas import tpu_sc as plsc

_LANES = 16


def _make_sc_kernel(B, SEQ):
    info = plsc.get_sparse_core_info()
    NC, NS = info.num_cores, info.num_subcores
    NW = NC * NS
    R = B // NW
    N = R * SEQ
    G = R // _LANES
    assert R % _LANES == 0 and B % NW == 0
    mesh = plsc.VectorSubcoreMesh(core_axis_name="c", subcore_axis_name="s")

    def _reduce(vals_v, out_v):
        iota_rows = lax.iota(jnp.int32, _LANES) * SEQ

        def body(j, accs):
            return tuple(
                accs[g]
                + plsc.load_gather(vals_v, [iota_rows + (g * _LANES * SEQ + j)])
                for g in range(G)
            )

        zeros = jnp.zeros((_LANES,), jnp.float32)
        accs = lax.fori_loop(0, SEQ, body, (zeros,) * G, unroll=2)
        for g in range(G):
            out_v[pl.ds(g * _LANES, _LANES)] = accs[g]

    @functools.partial(
        pl.kernel,
        out_type=jax.ShapeDtypeStruct((B,), jnp.float32),
        mesh=mesh,
        compiler_params=pltpu.CompilerParams(needs_layout_passes=False),
        scratch_types=dict(
            idx_v=pltpu.VMEM((N,), jnp.int32),
            vals_v=pltpu.VMEM((N,), jnp.float32),
            out_v=pltpu.VMEM((R,), jnp.float32),
            sem=pltpu.SemaphoreType.DMA,
        ),
    )
    def sc_kernel(idx_hbm, w_hbm, out_hbm, *, idx_v, vals_v, out_v, sem):
        wid = lax.axis_index("s") * NC + lax.axis_index("c")
        base = wid * N
        rbase = wid * R

        pltpu.sync_copy(idx_hbm.at[pl.ds(base, N)], idx_v)
        pltpu.async_copy(w_hbm.at[idx_v], vals_v, sem).wait()
        _reduce(vals_v, out_v)
        pltpu.sync_copy(out_v, out_hbm.at[pl.ds(rbase, R)])

    return sc_kernel


def kernel(inputs_pri, inputs_sec, W_pri, W_sec):
    B, SEQ = inputs_pri.shape
    sc = _make_sc_kernel(B, SEQ)
    out_sec = sc(inputs_sec.reshape(-1).astype(jnp.int32), W_sec[:, 0])
    out_pri = sc(inputs_pri.reshape(-1).astype(jnp.int32), W_pri[:, 0])
    return (out_pri, out_sec)


# speedup vs baseline: 1.0713x; 1.0057x over previous
import functools

import jax
import jax.numpy as jnp
from jax import lax
from jax.experimental import pallas as pl
from jax.experimental.pallas import tpu as pltpu
from jax.experimental.pallas import tpu_sc as plsc

_LANES = 16


def _make_sc_kernel(B, SEQ):
    info = plsc.get_sparse_core_info()
    NC, NS = info.num_cores, info.num_subcores
    NW = NC * NS
    R = B // NW
    N = R * SEQ
    G = R // _LANES
    assert R % _LANES == 0 and B % NW == 0
    mesh = plsc.VectorSubcoreMesh(core_axis_name="c", subcore_axis_name="s")

    def _reduce(vals_v, out_v, half):
        iota_rows = lax.iota(jnp.int32, _LANES) * SEQ
        gs = range(half * (G // 2), (half + 1) * (G // 2))

        def body(j, accs):
            return tuple(
                acc
                + plsc.load_gather(vals_v, [iota_rows + (g * _LANES * SEQ + j)])
                for acc, g in zip(accs, gs)
            )

        zeros = jnp.zeros((_LANES,), jnp.float32)
        accs = lax.fori_loop(0, SEQ, body, (zeros,) * (G // 2), unroll=2)
        for acc, g in zip(accs, gs):
            out_v[pl.ds(g * _LANES, _LANES)] = acc

    @functools.partial(
        pl.kernel,
        out_type=jax.ShapeDtypeStruct((B,), jnp.float32),
        mesh=mesh,
        compiler_params=pltpu.CompilerParams(needs_layout_passes=False),
        scratch_types=dict(
            idx_v=pltpu.VMEM((N,), jnp.int32),
            vals_v=pltpu.VMEM((N,), jnp.float32),
            out_v=pltpu.VMEM((R,), jnp.float32),
            sem_a=pltpu.SemaphoreType.DMA,
            sem_b=pltpu.SemaphoreType.DMA,
        ),
    )
    def sc_kernel(idx_hbm, w_hbm, out_hbm, *, idx_v, vals_v, out_v, sem_a, sem_b):
        wid = lax.axis_index("s") * NC + lax.axis_index("c")
        base = wid * N
        rbase = wid * R
        H = N // 2

        pltpu.sync_copy(idx_hbm.at[pl.ds(base, H)], idx_v.at[pl.ds(0, H)])
        ca = pltpu.async_copy(
            w_hbm.at[idx_v.at[pl.ds(0, H)]], vals_v.at[pl.ds(0, H)], sem_a
        )
        pltpu.sync_copy(idx_hbm.at[pl.ds(base + H, H)], idx_v.at[pl.ds(H, H)])
        cb = pltpu.async_copy(
            w_hbm.at[idx_v.at[pl.ds(H, H)]], vals_v.at[pl.ds(H, H)], sem_b
        )
        ca.wait()
        _reduce(vals_v, out_v, 0)
        cb.wait()
        _reduce(vals_v, out_v, 1)
        pltpu.sync_copy(out_v, out_hbm.at[pl.ds(rbase, R)])

    return sc_kernel


def kernel(inputs_pri, inputs_sec, W_pri, W_sec):
    B, SEQ = inputs_pri.shape
    sc = _make_sc_kernel(B, SEQ)
    out_sec = sc(inputs_sec.reshape(-1).astype(jnp.int32), W_sec[:, 0])
    out_pri = sc(inputs_pri.reshape(-1).astype(jnp.int32), W_pri[:, 0])
    return (out_pri, out_sec)


# speedup vs baseline: 1.1425x; 1.0665x over previous
import functools

import jax
import jax.numpy as jnp
from jax import lax
from jax.experimental import pallas as pl
from jax.experimental.pallas import tpu as pltpu
from jax.experimental.pallas import tpu_sc as plsc

_LANES = 16


def _make_sc_kernel(B, SEQ):
    info = plsc.get_sparse_core_info()
    NC, NS = info.num_cores, info.num_subcores
    NW = NC * NS
    R = B // NW
    N = R * SEQ
    G = R // _LANES
    assert R % _LANES == 0 and B % NW == 0
    mesh = plsc.VectorSubcoreMesh(core_axis_name="c", subcore_axis_name="s")

    def _reduce(vals_v, out_v, half):
        iota_rows = lax.iota(jnp.int32, _LANES) * SEQ
        gs = range(half * (G // 2), (half + 1) * (G // 2))

        def body(j, accs):
            return tuple(
                acc
                + plsc.load_gather(vals_v, [iota_rows + (g * _LANES * SEQ + j)])
                for acc, g in zip(accs, gs)
            )

        zeros = jnp.zeros((_LANES,), jnp.float32)
        accs = lax.fori_loop(0, SEQ, body, (zeros,) * (G // 2), unroll=2)
        for acc, g in zip(accs, gs):
            out_v[pl.ds(g * _LANES, _LANES)] = acc

    @functools.partial(
        pl.kernel,
        out_type=jax.ShapeDtypeStruct((B,), jnp.float32),
        mesh=mesh,
        compiler_params=pltpu.CompilerParams(needs_layout_passes=False),
        scratch_types=dict(
            idx_v=pltpu.VMEM((N,), jnp.int32),
            vals_v=pltpu.VMEM((N,), jnp.float32),
            out_v=pltpu.VMEM((R,), jnp.float32),
            sem_a=pltpu.SemaphoreType.DMA,
            sem_b=pltpu.SemaphoreType.DMA,
        ),
    )
    def sc_kernel(idx_hbm, w_hbm, out_hbm, *, idx_v, vals_v, out_v, sem_a, sem_b):
        wid = lax.axis_index("s") * NC + lax.axis_index("c")
        base = wid * N
        rbase = wid * R
        H = N // 2

        pltpu.sync_copy(idx_hbm.at[pl.ds(base, H)], idx_v.at[pl.ds(0, H)])
        ca = pltpu.async_copy(
            w_hbm.at[idx_v.at[pl.ds(0, H)]], vals_v.at[pl.ds(0, H)], sem_a
        )
        pltpu.sync_copy(idx_hbm.at[pl.ds(base + H, H)], idx_v.at[pl.ds(H, H)])
        cb = pltpu.async_copy(
            w_hbm.at[idx_v.at[pl.ds(H, H)]], vals_v.at[pl.ds(H, H)], sem_b
        )
        ca.wait()
        _reduce(vals_v, out_v, 0)
        cb.wait()
        _reduce(vals_v, out_v, 1)
        pltpu.sync_copy(out_v, out_hbm.at[pl.ds(rbase, R)])

    return sc_kernel


def kernel(inputs_pri, inputs_sec, W_pri, W_sec):
    B, SEQ = inputs_pri.shape
    sc = _make_sc_kernel(B, SEQ)
    out_sec = sc(inputs_sec.reshape(-1).astype(jnp.int32), W_sec[:, 0])
    anchor = (out_sec[0] * 0.0).astype(jnp.int32)
    out_pri = sc(
        inputs_pri.reshape(-1).astype(jnp.int32) + anchor, W_pri[:, 0]
    )
    return (out_pri, out_sec)
